# baseline (speedup 1.0000x reference)
import jax
import jax.numpy as jnp
from jax.experimental import pallas as pl
from jax.experimental.pallas import tpu as pltpu

_CHUNK = 256
_Q = 5
_QO = 3


def _gate_kernel(x_hbm, w_ref, out_hbm, x_buf, o_buf, wb_ref, in_sems, out_sems):
    n = x_hbm.shape[0] // _CHUNK

    half = _CHUNK // 2

    def in_copies(c, slot):
        return [
            pltpu.make_async_copy(
                x_hbm.at[pl.ds(c * _CHUNK + j * half, half), :],
                x_buf.at[slot, pl.ds(j * half, half), :],
                in_sems.at[slot],
            )
            for j in range(2)
        ]

    def out_copy(c, oslot):
        return pltpu.make_async_copy(
            o_buf.at[oslot],
            out_hbm.at[pl.ds(c * _CHUNK, _CHUNK), :],
            out_sems.at[oslot],
        )

    for c in range(_Q):
        for cp in in_copies(c, c):
            cp.start()

    wb_ref[...] = w_ref[...].astype(jnp.bfloat16)

    def body(c, _):
        slot = jax.lax.rem(c, _Q)
        oslot = jax.lax.rem(c, _QO)
        for cp in in_copies(c, slot):
            cp.wait()
        logits = jax.lax.dot_general(
            x_buf[slot].astype(jnp.bfloat16), wb_ref[...],
            dimension_numbers=(((1,), (1,)), ((), ())),
            preferred_element_type=jnp.float32)
        m = jnp.max(logits, axis=1, keepdims=True)
        e = jnp.exp(logits - m)
        gates = e / jnp.sum(e, axis=1, keepdims=True)

        @pl.when(c >= _QO)
        def _():
            out_copy(c - _QO, oslot).wait()

        o_buf[oslot] = gates
        out_copy(c, oslot).start()

        @pl.when(c + _Q < n)
        def _():
            for cp in in_copies(c + _Q, slot):
                cp.start()
        return 0

    jax.lax.fori_loop(0, n, body, 0)

    for t in range(_QO):
        c = n - _QO + t
        out_copy(c, jax.lax.rem(jnp.int32(c), _QO)).wait()


def kernel(inputs, W):
    tokens, d = inputs.shape
    n_exp = W.shape[0]
    return pl.pallas_call(
        _gate_kernel,
        in_specs=[
            pl.BlockSpec(memory_space=pl.ANY),
            pl.BlockSpec((n_exp, d), lambda: (0, 0)),
        ],
        out_specs=pl.BlockSpec(memory_space=pl.ANY),
        out_shape=jax.ShapeDtypeStruct((tokens, n_exp), jnp.float32),
        scratch_shapes=[
            pltpu.VMEM((_Q, _CHUNK, d), jnp.float32),
            pltpu.VMEM((_QO, _CHUNK, n_exp), jnp.float32),
            pltpu.VMEM((n_exp, d), jnp.bfloat16),
            pltpu.SemaphoreType.DMA((_Q,)),
            pltpu.SemaphoreType.DMA((_QO,)),
        ],
    )(inputs, W)


# speedup vs baseline: 1.0007x; 1.0007x over previous
import jax
import jax.numpy as jnp
from jax.experimental import pallas as pl
from jax.experimental.pallas import tpu as pltpu

_CHUNK = 256
_Q = 5
_QO = 3


def _gate_kernel(x_hbm, w_ref, out_hbm, x_buf, o_buf, wb_ref, in_sems, out_sems):
    n = x_hbm.shape[0] // _CHUNK

    def in_copy(c, slot):
        return pltpu.make_async_copy(
            x_hbm.at[pl.ds(c * _CHUNK, _CHUNK), :],
            x_buf.at[slot],
            in_sems.at[slot],
        )

    def out_copy(c, oslot):
        return pltpu.make_async_copy(
            o_buf.at[oslot],
            out_hbm.at[pl.ds(c * _CHUNK, _CHUNK), :],
            out_sems.at[oslot],
        )

    for c in range(_Q):
        in_copy(c, c).start()

    wb_ref[...] = w_ref[...].astype(jnp.bfloat16)

    def body(c, _):
        slot = jax.lax.rem(c, _Q)
        oslot = jax.lax.rem(c, _QO)
        in_copy(c, slot).wait()
        logits = jax.lax.dot_general(
            x_buf[slot].astype(jnp.bfloat16), wb_ref[...],
            dimension_numbers=(((1,), (1,)), ((), ())),
            preferred_element_type=jnp.float32)
        m = jnp.max(logits, axis=1, keepdims=True)
        e = jnp.exp(logits - m)
        gates = e / jnp.sum(e, axis=1, keepdims=True)

        @pl.when(c >= _QO)
        def _():
            out_copy(c - _QO, oslot).wait()

        o_buf[oslot] = gates
        out_copy(c, oslot).start()

        @pl.when(c + _Q < n)
        def _():
            in_copy(c + _Q, slot).start()
        return 0

    jax.lax.fori_loop(0, n, body, 0)

    for t in range(_QO):
        c = n - _QO + t
        out_copy(c, jax.lax.rem(jnp.int32(c), _QO)).wait()


def kernel(inputs, W):
    tokens, d = inputs.shape
    n_exp = W.shape[0]
    return pl.pallas_call(
        _gate_kernel,
        in_specs=[
            pl.BlockSpec(memory_space=pl.ANY),
            pl.BlockSpec((n_exp, d), lambda: (0, 0)),
        ],
        out_specs=pl.BlockSpec(memory_space=pl.ANY),
        out_shape=jax.ShapeDtypeStruct((tokens, n_exp), jnp.float32),
        scratch_shapes=[
            pltpu.VMEM((_Q, _CHUNK, d), jnp.float32),
            pltpu.VMEM((_QO, _CHUNK, n_exp), jnp.float32),
            pltpu.VMEM((n_exp, d), jnp.bfloat16),
            pltpu.SemaphoreType.DMA((_Q,)),
            pltpu.SemaphoreType.DMA((_QO,)),
        ],
    )(inputs, W)
